# speedup vs baseline: 3.0259x; 3.0259x over previous
import functools

import jax
import jax.numpy as jnp
from jax import lax
from jax.experimental import pallas as pl
from jax.experimental.pallas import tpu as pltpu
from jax.experimental.pallas import tpu_sc as plsc

NC = 2
NS = 16
NW = NC * NS
LANES = 16
D = 128
CHUNK = 80


def _edge_b_kernel(x_ref, w_ref, o_ref):
    o_ref[...] = jnp.dot(x_ref[...], w_ref[...],
                         preferred_element_type=jnp.float32)


def _node_g_kernel(h_ref, w_ref, o_ref):
    o_ref[...] = jnp.dot(h_ref[...], w_ref[...],
                         preferred_element_type=jnp.float32)


def _final_kernel(h_ref, agg_ref, w2_ref, o_ref):
    a = agg_ref[0] + agg_ref[1]
    y = (jnp.dot(h_ref[...], w2_ref[:D, :], preferred_element_type=jnp.float32)
         + jnp.dot(a, w2_ref[D:, :], preferred_element_type=jnp.float32))
    o_ref[...] = jnp.maximum(y, 0.0) + h_ref[...]


def _sc_edge_kernel(n_pad, epw, src_hbm, dst_hbm, g_hbm, b_hbm, out_hbm,
                    src_v, dst_v, g_v, b_v, agg_sh, sem):
    cid = lax.axis_index("c")
    sid = lax.axis_index("s")
    wid = sid * NC + cid
    rows_per_tile = n_pad // NS

    def zero_row(i, carry):
        for j in range(D // LANES):
            g_v[i, pl.ds(j * LANES, LANES)] = jnp.zeros((LANES,), jnp.float32)
        return carry
    lax.fori_loop(0, CHUNK, zero_row, 0)
    for j in range(rows_per_tile // CHUNK):
        pltpu.sync_copy(g_v, agg_sh.at[pl.ds(sid * rows_per_tile + j * CHUNK,
                                             CHUNK)])
    plsc.subcore_barrier()

    nchunks = epw // CHUNK

    def chunk_body(c, carry):
        base = pl.multiple_of(wid * epw + c * CHUNK, 8)
        pltpu.sync_copy(src_hbm.at[pl.ds(base, CHUNK)], src_v)
        pltpu.sync_copy(dst_hbm.at[pl.ds(base, CHUNK)], dst_v)
        gather = pltpu.async_copy(g_hbm.at[src_v], g_v, sem)
        pltpu.sync_copy(b_hbm.at[pl.ds(base, CHUNK)], b_v)
        gather.wait()

        def row_body(i, inner):
            for j in range(D // LANES):
                sl = pl.ds(j * LANES, LANES)
                g_v[i, sl] = jnp.maximum(g_v[i, sl] + b_v[i, sl], 0.0)
            return inner
        lax.fori_loop(0, CHUNK, row_body, 0)

        pltpu.sync_copy(g_v, agg_sh.at[dst_v], add=True)
        return carry

    lax.fori_loop(0, nchunks, chunk_body, 0)
    plsc.subcore_barrier()

    r0 = sid * rows_per_tile
    pltpu.sync_copy(agg_sh.at[pl.ds(r0, rows_per_tile)],
                    out_hbm.at[cid, pl.ds(r0, rows_per_tile)])


def kernel(H, idx, X_e, W1, W2):
    n_nodes = H.shape[0]
    n_edges = X_e.shape[0]
    epw = n_edges // NW
    n_pad = ((n_nodes + 639) // 640) * 640

    src = idx[0].astype(jnp.int32)
    dst = idx[1].astype(jnp.int32)

    g_mat = pl.pallas_call(
        _node_g_kernel,
        out_shape=jax.ShapeDtypeStruct((n_nodes, D), jnp.float32),
    )(H, W1[:D])

    eb = 4000
    b_mat = pl.pallas_call(
        _edge_b_kernel,
        grid=(n_edges // eb,),
        in_specs=[
            pl.BlockSpec((eb, X_e.shape[1]), lambda i: (i, 0)),
            pl.BlockSpec((X_e.shape[1], D), lambda i: (0, 0)),
        ],
        out_specs=pl.BlockSpec((eb, D), lambda i: (i, 0)),
        out_shape=jax.ShapeDtypeStruct((n_edges, D), jnp.float32),
    )(X_e, W1[D:])

    sc_edge = functools.partial(_sc_edge_kernel, n_pad, epw)
    agg2 = pl.kernel(
        sc_edge,
        out_type=jax.ShapeDtypeStruct((NC, n_pad, D), jnp.float32),
        mesh=plsc.VectorSubcoreMesh(core_axis_name="c", subcore_axis_name="s",
                                    num_cores=NC, num_subcores=NS),
        scratch_types=[
            pltpu.VMEM((CHUNK,), jnp.int32),
            pltpu.VMEM((CHUNK,), jnp.int32),
            pltpu.VMEM((CHUNK, D), jnp.float32),
            pltpu.VMEM((CHUNK, D), jnp.float32),
            pltpu.VMEM_SHARED((n_pad, D), jnp.float32),
            pltpu.SemaphoreType.DMA,
        ],
    )(src, dst, g_mat, b_mat)

    nb = 2000
    out = pl.pallas_call(
        _final_kernel,
        grid=(n_nodes // nb,),
        in_specs=[
            pl.BlockSpec((nb, D), lambda i: (i, 0)),
            pl.BlockSpec((NC, nb, D), lambda i: (0, i, 0)),
            pl.BlockSpec((2 * D, D), lambda i: (0, 0)),
        ],
        out_specs=pl.BlockSpec((nb, D), lambda i: (i, 0)),
        out_shape=jax.ShapeDtypeStruct((n_nodes, D), jnp.float32),
    )(H, agg2[:, :n_nodes], W2)

    return out
